# baseline (speedup 1.0000x reference)
import functools

import jax
import jax.numpy as jnp
from jax import lax
from jax.experimental import pallas as pl
from jax.experimental.pallas import tpu as pltpu
from jax.experimental.pallas import tpu_sc as plsc

D = 64
SCALE = 8.0
NC, NS = 2, 16
NW = NC * NS
CHUNK = 128
NBUF = 4
LANES = 16
DT = D // 8


PAD = D + 1


def _emb_kernel(n_chunks, n_cols, x_hbm, lut_hbm, out_hbm, idx_v, gbufs,
                tbufs, pbuf, gsems, ssems):
    wid = lax.axis_index("s") * NC + lax.axis_index("c")
    c0 = wid * n_chunks

    pltpu.sync_copy(x_hbm.at[wid], idx_v)

    lane65 = lax.iota(jnp.int32, LANES) * PAD

    def gather_start(j, b):
        pltpu.make_async_copy(lut_hbm.at[idx_v.at[j]], gbufs[b],
                              gsems[b]).start()

    def gather_wait(b):
        pltpu.make_async_copy(lut_hbm.at[idx_v.at[0]], gbufs[b],
                              gsems[b]).wait()

    def out_start(j, b):
        c = c0 + j
        b1 = c // n_cols
        b0t = c % n_cols
        for dt in range(DT):
            pltpu.make_async_copy(tbufs[b].at[dt],
                                  out_hbm.at[b1, dt, b0t], ssems[b]).start()

    def out_wait(b):
        for dt in range(DT):
            pltpu.make_async_copy(tbufs[b].at[dt], out_hbm.at[0, dt, 0],
                                  ssems[b]).wait()

    def transpose_scale(b):
        src, dst = gbufs[b], tbufs[b]

        @plsc.parallel_loop(0, CHUNK // LANES)
        def _(q):
            for jj in range(LANES):
                j = q * LANES + jj
                for k in range(D // LANES):
                    pbuf[pl.ds(j * PAD + k * LANES, LANES)] = (
                        src[j, pl.ds(k * LANES, LANES)] * SCALE)
            rows = lane65 + q * (LANES * PAD)
            rp = [rows + r for r in range(8)]
            for d in range(D):
                base = (d // 8) * 8
                v = plsc.load_gather(
                    pbuf.at[pl.ds(base, CHUNK * PAD - (DT - 1) * 8)],
                    [rp[d % 8]])
                dst[d // 8, d % 8, pl.ds(q * LANES, LANES)] = v

    for b in range(NBUF):
        gather_start(b, b)

    def step(g, carry):
        for b in range(NBUF):
            j = g * NBUF + b
            gather_wait(b)

            @pl.when(g > 0)
            def _():
                out_wait(b)

            transpose_scale(b)
            out_start(j, b)

            @pl.when(j + NBUF < n_chunks)
            def _():
                gather_start(j + NBUF, b)

        return carry

    lax.fori_loop(0, n_chunks // NBUF, step, 0)

    for b in range(NBUF):
        out_wait(b)


def kernel(x, lut):
    b0, b1 = x.shape
    n_cols = b0 // CHUNK
    n_total = b1 * n_cols
    n_chunks = n_total // NW
    x_t = jnp.swapaxes(x, 0, 1).astype(jnp.int32).reshape(NW, n_chunks, CHUNK)

    mesh = plsc.VectorSubcoreMesh(core_axis_name="c", subcore_axis_name="s",
                                  num_cores=NC, num_subcores=NS)
    run = functools.partial(
        pl.kernel,
        out_type=jax.ShapeDtypeStruct((b1, DT, n_cols, 8, CHUNK),
                                      jnp.float32),
        mesh=mesh,
        compiler_params=pltpu.CompilerParams(use_tc_tiling_on_sc=False,
                                             needs_layout_passes=False,
                                             disable_bounds_checks=True),
        scratch_types=[
            pltpu.VMEM((n_chunks, CHUNK), jnp.int32),
            [pltpu.VMEM((CHUNK, D), jnp.float32) for _ in range(NBUF)],
            [pltpu.VMEM((DT, 8, CHUNK), jnp.float32) for _ in range(NBUF)],
            pltpu.VMEM((CHUNK * PAD,), jnp.float32),
            [pltpu.SemaphoreType.DMA for _ in range(NBUF)],
            [pltpu.SemaphoreType.DMA for _ in range(NBUF)],
        ],
    )(functools.partial(_emb_kernel, n_chunks, n_cols))
    buf = run(x_t, lut)
    return buf.transpose(2, 4, 0, 1, 3).reshape(b0, b1, D)


# speedup vs baseline: 1.2841x; 1.2841x over previous
import functools

import jax
import jax.numpy as jnp
from jax import lax
from jax.experimental import pallas as pl
from jax.experimental.pallas import tpu as pltpu
from jax.experimental.pallas import tpu_sc as plsc

D = 64
SCALE = 8.0
NC, NS = 2, 16
NW = NC * NS
CHUNK = 128
NBUF = 2
LANES = 16
DT = D // 8


PAD = D + 1


def _emb_kernel(n_chunks, n_cols, x_hbm, lut_hbm, out_hbm, idx_v, gbufs,
                tbufs, pbuf, gsems, ssems):
    wid = lax.axis_index("s") * NC + lax.axis_index("c")
    c0 = wid * n_chunks

    pltpu.sync_copy(x_hbm.at[wid], idx_v)

    lane65 = lax.iota(jnp.int32, LANES) * PAD

    def gather_start(j, b):
        pltpu.make_async_copy(lut_hbm.at[idx_v.at[j]], gbufs[b],
                              gsems[b]).start()

    def gather_wait(b):
        pltpu.make_async_copy(lut_hbm.at[idx_v.at[0]], gbufs[b],
                              gsems[b]).wait()

    def out_start(j, b):
        c = c0 + j
        b1 = c // n_cols
        b0t = c % n_cols
        for dt in range(DT):
            pltpu.make_async_copy(tbufs[b].at[dt],
                                  out_hbm.at[b1, dt, b0t], ssems[b]).start()

    def out_wait(b):
        for dt in range(DT):
            pltpu.make_async_copy(tbufs[b].at[dt], out_hbm.at[0, dt, 0],
                                  ssems[b]).wait()

    def transpose_scale(b):
        src, dst = gbufs[b], tbufs[b]

        @plsc.parallel_loop(0, CHUNK, unroll=4)
        def _(j):
            for k in range(D // LANES):
                pbuf[pl.ds(j * PAD + k * LANES, LANES)] = (
                    src[j, pl.ds(k * LANES, LANES)] * SCALE)

        @plsc.parallel_loop(0, CHUNK // LANES)
        def _(q):
            rows = lane65 + q * (LANES * PAD)
            rp = [rows + r for r in range(8)]
            for d in range(D):
                base = (d // 8) * 8
                v = plsc.load_gather(
                    pbuf.at[pl.ds(base, CHUNK * PAD - (DT - 1) * 8)],
                    [rp[d % 8]])
                dst[d // 8, d % 8, pl.ds(q * LANES, LANES)] = v

    for b in range(NBUF):
        gather_start(b, b)

    def step(g, carry):
        for b in range(NBUF):
            j = g * NBUF + b
            gather_wait(b)

            @pl.when(g > 0)
            def _():
                out_wait(b)

            transpose_scale(b)
            out_start(j, b)

            @pl.when(j + NBUF < n_chunks)
            def _():
                gather_start(j + NBUF, b)

        return carry

    lax.fori_loop(0, n_chunks // NBUF, step, 0)

    for b in range(NBUF):
        out_wait(b)


def kernel(x, lut):
    b0, b1 = x.shape
    n_cols = b0 // CHUNK
    n_total = b1 * n_cols
    n_chunks = n_total // NW
    x_t = jnp.swapaxes(x, 0, 1).astype(jnp.int32).reshape(NW, n_chunks, CHUNK)

    mesh = plsc.VectorSubcoreMesh(core_axis_name="c", subcore_axis_name="s",
                                  num_cores=NC, num_subcores=NS)
    run = functools.partial(
        pl.kernel,
        out_type=jax.ShapeDtypeStruct((b1, DT, n_cols, 8, CHUNK),
                                      jnp.float32),
        mesh=mesh,
        compiler_params=pltpu.CompilerParams(use_tc_tiling_on_sc=False,
                                             needs_layout_passes=False,
                                             disable_bounds_checks=True),
        scratch_types=[
            pltpu.VMEM((n_chunks, CHUNK), jnp.int32),
            [pltpu.VMEM((CHUNK, D), jnp.float32) for _ in range(NBUF)],
            [pltpu.VMEM((DT, 8, CHUNK), jnp.float32) for _ in range(NBUF)],
            pltpu.VMEM((CHUNK * PAD,), jnp.float32),
            [pltpu.SemaphoreType.DMA for _ in range(NBUF)],
            [pltpu.SemaphoreType.DMA for _ in range(NBUF)],
        ],
    )(functools.partial(_emb_kernel, n_chunks, n_cols))
    buf = run(x_t, lut)
    return buf.transpose(2, 4, 0, 1, 3).reshape(b0, b1, D)


# speedup vs baseline: 1.2890x; 1.0038x over previous
import functools

import jax
import jax.numpy as jnp
from jax import lax
from jax.experimental import pallas as pl
from jax.experimental.pallas import tpu as pltpu
from jax.experimental.pallas import tpu_sc as plsc

D = 64
SCALE = 8.0
NC, NS = 2, 16
NW = NC * NS
CHUNK = 128
NBUF = 2
LANES = 16
DT = D // 8


PAD = D + 1


def _emb_kernel(n_chunks, n_cols, x_hbm, lut_hbm, out_hbm, idx_v, gbufs,
                tbufs, pbuf, gsems, ssems):
    wid = lax.axis_index("s") * NC + lax.axis_index("c")
    c0 = wid * n_chunks

    pltpu.sync_copy(x_hbm.at[wid], idx_v)

    lane65 = lax.iota(jnp.int32, LANES) * PAD

    def gather_start(j, b):
        pltpu.make_async_copy(lut_hbm.at[idx_v.at[j]], gbufs[b],
                              gsems[b]).start()

    def gather_wait(b):
        pltpu.make_async_copy(lut_hbm.at[idx_v.at[0]], gbufs[b],
                              gsems[b]).wait()

    def out_start(j, b):
        c = c0 + j
        b1 = c // n_cols
        b0t = c % n_cols
        for dt in range(DT):
            pltpu.make_async_copy(tbufs[b].at[dt],
                                  out_hbm.at[b1, dt, b0t], ssems[b]).start()

    def out_wait(b):
        for dt in range(DT):
            pltpu.make_async_copy(tbufs[b].at[dt], out_hbm.at[0, dt, 0],
                                  ssems[b]).wait()

    def transpose_scale(b):
        src, dst = gbufs[b], tbufs[b]

        @plsc.parallel_loop(0, CHUNK, unroll=8)
        def _(j):
            for k in range(D // LANES):
                pbuf[pl.ds(j * PAD + k * LANES, LANES)] = (
                    src[j, pl.ds(k * LANES, LANES)] * SCALE)

        @plsc.parallel_loop(0, CHUNK // LANES)
        def _(q):
            rows = lane65 + q * (LANES * PAD)
            rp = [rows + r for r in range(8)]
            for d in range(D):
                base = (d // 8) * 8
                v = plsc.load_gather(
                    pbuf.at[pl.ds(base, CHUNK * PAD - (DT - 1) * 8)],
                    [rp[d % 8]])
                dst[d // 8, d % 8, pl.ds(q * LANES, LANES)] = v

    for b in range(NBUF):
        gather_start(b, b)

    def step(g, carry):
        for b in range(NBUF):
            j = g * NBUF + b
            gather_wait(b)

            @pl.when(g > 0)
            def _():
                out_wait(b)

            transpose_scale(b)
            out_start(j, b)

            @pl.when(j + NBUF < n_chunks)
            def _():
                gather_start(j + NBUF, b)

        return carry

    lax.fori_loop(0, n_chunks // NBUF, step, 0)

    for b in range(NBUF):
        out_wait(b)


def kernel(x, lut):
    b0, b1 = x.shape
    n_cols = b0 // CHUNK
    n_total = b1 * n_cols
    n_chunks = n_total // NW
    x_t = jnp.swapaxes(x, 0, 1).astype(jnp.int32).reshape(NW, n_chunks, CHUNK)

    mesh = plsc.VectorSubcoreMesh(core_axis_name="c", subcore_axis_name="s",
                                  num_cores=NC, num_subcores=NS)
    run = functools.partial(
        pl.kernel,
        out_type=jax.ShapeDtypeStruct((b1, DT, n_cols, 8, CHUNK),
                                      jnp.float32),
        mesh=mesh,
        compiler_params=pltpu.CompilerParams(use_tc_tiling_on_sc=False,
                                             needs_layout_passes=False,
                                             disable_bounds_checks=True),
        scratch_types=[
            pltpu.VMEM((n_chunks, CHUNK), jnp.int32),
            [pltpu.VMEM((CHUNK, D), jnp.float32) for _ in range(NBUF)],
            [pltpu.VMEM((DT, 8, CHUNK), jnp.float32) for _ in range(NBUF)],
            pltpu.VMEM((CHUNK * PAD,), jnp.float32),
            [pltpu.SemaphoreType.DMA for _ in range(NBUF)],
            [pltpu.SemaphoreType.DMA for _ in range(NBUF)],
        ],
    )(functools.partial(_emb_kernel, n_chunks, n_cols))
    buf = run(x_t, lut)
    return buf.transpose(2, 4, 0, 1, 3).reshape(b0, b1, D)


# speedup vs baseline: 1.3773x; 1.0685x over previous
import functools

import jax
import jax.numpy as jnp
from jax import lax
from jax.experimental import pallas as pl
from jax.experimental.pallas import tpu as pltpu
from jax.experimental.pallas import tpu_sc as plsc

D = 64
SCALE = 8.0
NC, NS = 2, 16
NW = NC * NS
CHUNK = 128
NBUF = 2
LANES = 16
DT = D // 8


PAD = D + 1


def _emb_kernel(n_chunks, n_cols, x_hbm, lut_hbm, out_hbm, idx_v, gbufs,
                tbufs, pbuf, gsems, ssems):
    wid = lax.axis_index("s") * NC + lax.axis_index("c")
    c0 = wid * n_chunks

    pltpu.sync_copy(x_hbm.at[wid], idx_v)

    lane65 = lax.iota(jnp.int32, LANES) * PAD

    def gather_start(j, b):
        pltpu.make_async_copy(lut_hbm.at[idx_v.at[j]], gbufs[b],
                              gsems[b]).start()

    def gather_wait(b):
        pltpu.make_async_copy(lut_hbm.at[idx_v.at[0]], gbufs[b],
                              gsems[b]).wait()

    def out_start(j, b):
        c = c0 + j
        b1 = c // n_cols
        b0t = c % n_cols
        pltpu.make_async_copy(tbufs[b], out_hbm.at[b1, :, b0t],
                              ssems[b]).start()

    def out_wait(b):
        pltpu.make_async_copy(tbufs[b], out_hbm.at[0, :, 0],
                              ssems[b]).wait()

    def transpose_scale(b):
        src, dst = gbufs[b], tbufs[b]

        @plsc.parallel_loop(0, CHUNK, unroll=8)
        def _(j):
            for k in range(D // LANES):
                pbuf[pl.ds(j * PAD + k * LANES, LANES)] = (
                    src[j, pl.ds(k * LANES, LANES)] * SCALE)

        @plsc.parallel_loop(0, CHUNK // LANES)
        def _(q):
            rows = lane65 + q * (LANES * PAD)
            rp = [rows + r for r in range(8)]
            for d in range(D):
                base = (d // 8) * 8
                v = plsc.load_gather(
                    pbuf.at[pl.ds(base, CHUNK * PAD - (DT - 1) * 8)],
                    [rp[d % 8]])
                dst[d // 8, d % 8, pl.ds(q * LANES, LANES)] = v

    for b in range(NBUF):
        gather_start(b, b)

    def step(g, carry):
        for b in range(NBUF):
            j = g * NBUF + b
            gather_wait(b)

            @pl.when(g > 0)
            def _():
                out_wait(b)

            transpose_scale(b)
            out_start(j, b)

            @pl.when(j + NBUF < n_chunks)
            def _():
                gather_start(j + NBUF, b)

        return carry

    lax.fori_loop(0, n_chunks // NBUF, step, 0)

    for b in range(NBUF):
        out_wait(b)


def kernel(x, lut):
    b0, b1 = x.shape
    n_cols = b0 // CHUNK
    n_total = b1 * n_cols
    n_chunks = n_total // NW
    x_t = jnp.swapaxes(x, 0, 1).astype(jnp.int32).reshape(NW, n_chunks, CHUNK)

    mesh = plsc.VectorSubcoreMesh(core_axis_name="c", subcore_axis_name="s",
                                  num_cores=NC, num_subcores=NS)
    run = functools.partial(
        pl.kernel,
        out_type=jax.ShapeDtypeStruct((b1, DT, n_cols, 8, CHUNK),
                                      jnp.float32),
        mesh=mesh,
        compiler_params=pltpu.CompilerParams(use_tc_tiling_on_sc=False,
                                             needs_layout_passes=False,
                                             disable_bounds_checks=True),
        scratch_types=[
            pltpu.VMEM((n_chunks, CHUNK), jnp.int32),
            [pltpu.VMEM((CHUNK, D), jnp.float32) for _ in range(NBUF)],
            [pltpu.VMEM((DT, 8, CHUNK), jnp.float32) for _ in range(NBUF)],
            pltpu.VMEM((CHUNK * PAD,), jnp.float32),
            [pltpu.SemaphoreType.DMA for _ in range(NBUF)],
            [pltpu.SemaphoreType.DMA for _ in range(NBUF)],
        ],
    )(functools.partial(_emb_kernel, n_chunks, n_cols))
    buf = run(x_t, lut)
    return buf.transpose(2, 4, 0, 1, 3).reshape(b0, b1, D)
